# Initial kernel scaffold; baseline (speedup 1.0000x reference)
import jax
import jax.numpy as jnp
from jax.experimental import pallas as pl


def kernel(z, codebook):
    raise NotImplementedError("write your pallas kernel here")


# speedup vs baseline: 1.1751x; 1.1751x over previous
import jax
import jax.numpy as jnp
from jax.experimental import pallas as pl

_BETA = 0.25
_N_TOK = 2048
_CODE_DIM = 256
_K = 1024
_BLK = 256


def _vq_block(z_ref, c_ref, zq_ref, part_ref):
    z = z_ref[...]
    c = c_ref[...]
    m = jnp.dot(z, c.T, preferred_element_type=jnp.float32)
    z2 = jnp.sum(z * z, axis=1, keepdims=True)
    c2 = jnp.sum(c * c, axis=1)[None, :]
    dist = z2 - 2.0 * m + c2
    rowmin = jnp.min(dist, axis=1, keepdims=True)
    iota = jax.lax.broadcasted_iota(jnp.int32, dist.shape, 1)
    idx = jnp.min(jnp.where(dist == rowmin, iota, _K), axis=1,
                  keepdims=True)
    onehot = (iota == idx).astype(jnp.float32)
    zq = jnp.dot(onehot, c, preferred_element_type=jnp.float32)
    zq_ref[...] = zq
    part = jnp.sum((zq - z) ** 2)
    part_ref[...] = jnp.full((1, 1, 128), part, jnp.float32)


def kernel(z, codebook):
    z = z.reshape(z.shape[0], -1)
    zq, parts = pl.pallas_call(
        _vq_block,
        grid=(_N_TOK // _BLK,),
        in_specs=[
            pl.BlockSpec((_BLK, _CODE_DIM), lambda i: (i, 0)),
            pl.BlockSpec((_K, _CODE_DIM), lambda i: (0, 0)),
        ],
        out_specs=[
            pl.BlockSpec((_BLK, _CODE_DIM), lambda i: (i, 0)),
            pl.BlockSpec((1, 1, 128), lambda i: (i, 0, 0)),
        ],
        out_shape=[
            jax.ShapeDtypeStruct((_N_TOK, _CODE_DIM), jnp.float32),
            jax.ShapeDtypeStruct((_N_TOK // _BLK, 1, 128), jnp.float32),
        ],
    )(z, codebook)
    mean_sq = jnp.sum(parts[:, 0, 0]) / (_N_TOK * _CODE_DIM)
    loss = _BETA * mean_sq + mean_sq
    return (zq, loss)
